# baseline (speedup 1.0000x reference)
import functools
import math

import jax
import jax.numpy as jnp
from jax.experimental import pallas as pl
from jax.experimental.pallas import tpu as pltpu

_THRESHOLD = 0.1
_BLK = 512


def _mask_kernel(ego_ref, cav_ref, f_ref, *, s_total):
    ego_b16 = ego_ref[0, 0].astype(jnp.bfloat16)
    cav_b16 = cav_ref[0, 0].astype(jnp.bfloat16)
    sqrt_c = jnp.float32(math.sqrt(2.0))
    rows = jax.lax.broadcasted_iota(jnp.int32, (_BLK, _BLK), 0)
    cols = jax.lax.broadcasted_iota(jnp.int32, (_BLK, _BLK), 1)
    eye_b16 = (rows == cols).astype(jnp.bfloat16)

    for r in range(s_total // _BLK):
        p0 = r * _BLK
        ab_cols = jax.lax.dot_general(
            eye_b16, ego_b16[:, p0:p0 + _BLK], (((1,), (1,)), ((), ())),
            preferred_element_type=jnp.float32)
        sim = jax.lax.dot_general(
            ab_cols.astype(jnp.bfloat16), cav_b16, (((1,), (0,)), ((), ())),
            preferred_element_type=jnp.float32) / sqrt_c
        m = jnp.max(sim, axis=1, keepdims=True)
        e = jnp.exp(sim - m)
        den = jnp.sum(e, axis=1, keepdims=True)
        attn_b16 = (e / den).astype(jnp.bfloat16)
        y = jax.lax.dot_general(
            attn_b16, cav_b16, (((1,), (1,)), ((), ())),
            preferred_element_type=jnp.float32)
        z = jnp.max(y, axis=1, keepdims=True)
        f_ref[0, 0, p0:p0 + _BLK, 0:1] = jax.nn.sigmoid(z)


def _threshold_kernel(f_ref, gate_ref, mask_ref, *, k):
    f = f_ref[0, 0]
    keys = jax.lax.bitcast_convert_type(f, jnp.int32)
    t = jnp.int32(0)
    for bit in range(30, -1, -1):
        cand = t | jnp.int32(1 << bit)
        cnt = jnp.sum((keys >= cand).astype(jnp.int32))
        t = jnp.where(cnt >= k, cand, t)
    g = gate_ref[0, 0, 0, 0]
    mask_ref[0, 0] = (keys >= t).astype(jnp.float32) * g


def _compute_masks(psm, gate):
    B, L, C2, H, W = psm.shape
    S = H * W
    psm_r = psm.reshape(B, L, C2, S)
    k = max(1, int(S * _THRESHOLD))
    kern = functools.partial(_mask_kernel, s_total=S)
    f = pl.pallas_call(
        kern,
        grid=(B, L - 1),
        in_specs=[
            pl.BlockSpec((1, 1, C2, S), lambda b, j: (b, 0, 0, 0)),
            pl.BlockSpec((1, 1, C2, S), lambda b, j: (b, j + 1, 0, 0)),
        ],
        out_specs=pl.BlockSpec((1, 1, S, 1), lambda b, j: (b, j, 0, 0)),
        out_shape=jax.ShapeDtypeStruct((B, L - 1, S, 1), jnp.float32),
        compiler_params=pltpu.CompilerParams(
            dimension_semantics=("parallel", "parallel")),
    )(psm_r, psm_r)
    return pl.pallas_call(
        functools.partial(_threshold_kernel, k=k),
        grid=(B, L - 1),
        in_specs=[
            pl.BlockSpec((1, 1, 1, S), lambda b, j: (b, j, 0, 0)),
            pl.BlockSpec((1, 1, 1, 1), lambda b, j: (b, j, 0, 0)),
        ],
        out_specs=pl.BlockSpec((1, 1, 1, S), lambda b, j: (b, j, 0, 0)),
        out_shape=jax.ShapeDtypeStruct((B, L - 1, 1, S), jnp.float32),
        compiler_params=pltpu.CompilerParams(
            dimension_semantics=("parallel", "parallel")),
    )(f.reshape(B, L - 1, 1, S), gate)


def _apply_kernel(x_ref, m_ref, o_ref):
    @pl.when(pl.program_id(1) == 0)
    def _copy():
        o_ref[...] = x_ref[...]

    @pl.when(pl.program_id(1) != 0)
    def _mask():
        o_ref[...] = x_ref[...] * m_ref[...]


def kernel(x, psm, mask):
    B, L, C, H, W = x.shape
    S = H * W
    gate = (mask[:, 1:] != 0).astype(jnp.float32).reshape(B, L - 1, 1, 1)
    masks = _compute_masks(psm, gate)
    xr = x.reshape(B, L, C, S)
    cb = 64
    out = pl.pallas_call(
        _apply_kernel,
        grid=(B, L, C // cb),
        in_specs=[
            pl.BlockSpec((1, 1, cb, S), lambda b, l, c: (b, l, c, 0)),
            pl.BlockSpec((1, 1, 1, S),
                         lambda b, l, c: (b, jnp.maximum(l - 1, 0), 0, 0)),
        ],
        out_specs=pl.BlockSpec((1, 1, cb, S), lambda b, l, c: (b, l, c, 0)),
        out_shape=jax.ShapeDtypeStruct((B, L, C, S), x.dtype),
        compiler_params=pltpu.CompilerParams(
            dimension_semantics=("parallel", "parallel", "parallel")),
    )(xr, masks)
    return out.reshape(B, L, C, H, W)


# speedup vs baseline: 1.1601x; 1.1601x over previous
import functools
import math

import jax
import jax.numpy as jnp
from jax.experimental import pallas as pl
from jax.experimental.pallas import tpu as pltpu

_THRESHOLD = 0.1
_BLK = 256


def _mask_kernel(ego_ref, cav_ref, f_ref, *, s_total):
    ego_b16 = ego_ref[0, 0].astype(jnp.bfloat16)
    cav_b16 = cav_ref[0, 0].astype(jnp.bfloat16)
    u_row = cav_b16[0:1, :].astype(jnp.float32)
    v_row = cav_b16[1:2, :].astype(jnp.float32)
    sqrt_c = jnp.float32(math.sqrt(2.0))
    rows = jax.lax.broadcasted_iota(jnp.int32, (_BLK, _BLK), 0)
    cols = jax.lax.broadcasted_iota(jnp.int32, (_BLK, _BLK), 1)
    eye_b16 = (rows == cols).astype(jnp.bfloat16)

    for r in range(s_total // _BLK):
        p0 = r * _BLK
        ab_cols = jax.lax.dot_general(
            eye_b16, ego_b16[:, p0:p0 + _BLK], (((1,), (1,)), ((), ())),
            preferred_element_type=jnp.float32)
        a_col = ab_cols[:, 0:1]
        b_col = ab_cols[:, 1:2]
        sim = (a_col * u_row + b_col * v_row) / sqrt_c
        m = jnp.max(sim, axis=1, keepdims=True)
        e = jnp.exp(sim - m)
        den = jnp.sum(e, axis=1, keepdims=True)
        attn_b16 = (e / den).astype(jnp.bfloat16)
        y = jax.lax.dot_general(
            attn_b16, cav_b16, (((1,), (1,)), ((), ())),
            preferred_element_type=jnp.float32)
        z = jnp.max(y, axis=1, keepdims=True)
        f_ref[0, 0, p0:p0 + _BLK, 0:1] = jax.nn.sigmoid(z)


def _compute_f(psm):
    B, L, C2, H, W = psm.shape
    S = H * W
    psm_r = psm.reshape(B, L, C2, S)
    kern = functools.partial(_mask_kernel, s_total=S)
    f = pl.pallas_call(
        kern,
        grid=(B, L - 1),
        in_specs=[
            pl.BlockSpec((1, 1, C2, S), lambda b, j: (b, 0, 0, 0)),
            pl.BlockSpec((1, 1, C2, S), lambda b, j: (b, j + 1, 0, 0)),
        ],
        out_specs=pl.BlockSpec((1, 1, S, 1), lambda b, j: (b, j, 0, 0)),
        out_shape=jax.ShapeDtypeStruct((B, L - 1, S, 1), jnp.float32),
        compiler_params=pltpu.CompilerParams(
            dimension_semantics=("parallel", "parallel")),
    )(psm_r, psm_r)
    return f.reshape(B, L - 1, 1, S)


def _apply_kernel(f_ref, gate_ref, x_ref, o_ref, m_sc, *, k):
    l = pl.program_id(1)
    c = pl.program_id(2)

    @pl.when((l != 0) & (c == 0))
    def _threshold():
        f = f_ref[0, 0]
        keys = jax.lax.bitcast_convert_type(f, jnp.int32)
        t = jnp.zeros((1, 1), jnp.int32)
        for bit in range(30, -1, -1):
            cand = t | jnp.int32(1 << bit)
            cnt = jnp.sum((keys >= cand).astype(jnp.int32), axis=1,
                          keepdims=True)
            t = jnp.where(cnt >= k, cand, t)
        g = gate_ref[0, 0, 0, 0]
        m_sc[0:1, :] = (keys >= t).astype(jnp.float32) * g

    @pl.when(l == 0)
    def _copy():
        o_ref[...] = x_ref[...]

    @pl.when(l != 0)
    def _mask():
        o_ref[...] = x_ref[...] * m_sc[0:1, :][None, None]


def kernel(x, psm, mask):
    B, L, C, H, W = x.shape
    S = H * W
    gate = (mask[:, 1:] != 0).astype(jnp.float32).reshape(B, L - 1, 1, 1)
    f = _compute_f(psm)
    xr = x.reshape(B, L, C, S)
    cb = 128
    k = max(1, int(S * _THRESHOLD))
    out = pl.pallas_call(
        functools.partial(_apply_kernel, k=k),
        grid=(B, L, C // cb),
        in_specs=[
            pl.BlockSpec((1, 1, 1, S),
                         lambda b, l, c: (b, jnp.maximum(l - 1, 0), 0, 0)),
            pl.BlockSpec((1, 1, 1, 1),
                         lambda b, l, c: (b, jnp.maximum(l - 1, 0), 0, 0)),
            pl.BlockSpec((1, 1, cb, S), lambda b, l, c: (b, l, c, 0)),
        ],
        out_specs=pl.BlockSpec((1, 1, cb, S), lambda b, l, c: (b, l, c, 0)),
        out_shape=jax.ShapeDtypeStruct((B, L, C, S), x.dtype),
        scratch_shapes=[pltpu.VMEM((1, S), jnp.float32)],
    )(f, gate, xr)
    return out.reshape(B, L, C, H, W)
